# baseline (speedup 1.0000x reference)
import functools

import jax
import jax.numpy as jnp
from jax import lax
from jax.experimental import pallas as pl
from jax.experimental.pallas import tpu as pltpu
from jax.experimental.pallas import tpu_sc as plsc

B = 16
N = 16384
M = 64
L = 16
NC = 2
NS = 16
NW = NC * NS
TRW = B * N // 2
SPW = TRW // 16
CH = 2048
NCHUNK = SPW // CH
GRP = CH // L
GT = CH // 1024

_mesh = plsc.VectorSubcoreMesh(core_axis_name="c", subcore_axis_name="s")


def _body(face_h, lmp_h, gzp_h, tbox_h, tlm_h, tgz_h, mat_h, lab_h, out_h,
          face_v0, face_v1, lmp_v0, lmp_v1, gzp_v0, gzp_v1,
          mat_v0, mat_v1, lab_v0, lab_v1,
          tbox_v, tlm_v, tgz_v, out_v, sem0, sem1):
    cid = lax.axis_index("c")
    sid = lax.axis_index("s")
    wid = sid * NC + cid
    tr = wid // 16
    stripe = wid % 16
    w0 = stripe * SPW

    pltpu.sync_copy(tbox_h.at[pl.ds(tr * 8, 8)], tbox_v)
    pltpu.sync_copy(tlm_h.at[pl.ds(tr * 8, 8)], tlm_v)
    pltpu.sync_copy(tgz_h.at[pl.ds(tr * 8, 8)], tgz_v)

    bufs = ((face_v0, lmp_v0, gzp_v0, mat_v0, lab_v0, sem0),
            (face_v1, lmp_v1, gzp_v1, mat_v1, lab_v1, sem1))

    def start(c, slot):
        fv, lv, gv, mv, bv, sem = bufs[slot]
        base = w0 + c * CH
        gz0 = (base // 1024) * 256
        hs = []
        for j in range(4):
            hs.append(pltpu.async_copy(
                face_h.at[j, tr, pl.ds(base, CH)], fv.at[pl.ds(j * CH, CH)], sem))
        for j in range(10):
            hs.append(pltpu.async_copy(
                lmp_h.at[j, tr, pl.ds(base, CH)], lv.at[pl.ds(j * CH, CH)], sem))
        hs.append(pltpu.async_copy(
            gzp_h.at[pl.ds(tr * 8, 8), pl.ds(gz0, GT * 256)], gv, sem))
        hs.append(pltpu.async_copy(mat_h.at[tr, pl.ds(base, CH)], mv, sem))
        hs.append(pltpu.async_copy(lab_h.at[tr, pl.ds(base, CH)], bv, sem))
        return hs

    iota = jnp.arange(L, dtype=jnp.int32)
    izero = iota * 0
    if5 = [iota + j * CH for j in range(4)]
    if10 = [iota + j * CH for j in range(10)]
    it4 = [izero + j * M for j in range(4)]
    it10 = [izero + j * M for j in range(10)]
    it2 = [izero + j * M for j in range(2)]
    ig2 = [iota + j * 128 for j in range(2)]

    def compute(slot, accs):
        fv, lv, gv, mv, bv, _ = bufs[slot]

        def group(g, accs):
            abox, alm, agz = accs
            off = g * L
            row = (g // 8) % 8
            fvo = fv.at[pl.ds(off, L)]
            lvo = lv.at[pl.ds(off, L)]
            m = plsc.load_gather(mv.at[pl.ds(off, L)], [iota])
            lab = plsc.load_gather(bv.at[pl.ds(off, L)], [iota])
            maskf = jnp.where(lab > 0.0, 1.0, 0.0).astype(jnp.float32)

            def sl1(pred_sl, p_bases, tbl_ref, t_bases, ncomp):
                sq = jnp.zeros((L,), jnp.float32)
                mx = jnp.zeros((L,), jnp.float32)
                for j in range(ncomp):
                    p = plsc.load_gather(pred_sl, [p_bases[j]])
                    t = plsc.load_gather(tbl_ref, [m + t_bases[j]])
                    d = jnp.abs(p - t)
                    dm = jnp.minimum(d, 1.0)
                    sq = sq + dm * dm
                    mx = mx + jnp.maximum(d, 1.0)
                return (0.5 * sq + mx - float(ncomp)) * maskf

            abox = abox + sl1(fvo, if5, tbox_v.at[row], it4, 4)
            alm = alm + sl1(lvo, if10, tlm_v.at[row], it10, 10)

            gb = (g // 64) * 256 + (g % 8) * L
            gvo = gv.at[row, pl.ds(gb, L)]
            sq = jnp.zeros((L,), jnp.float32)
            mx = jnp.zeros((L,), jnp.float32)
            for j in range(2):
                p = plsc.load_gather(gvo, [ig2[j]])
                t = plsc.load_gather(tgz_v.at[row], [m + it2[j]])
                d = jnp.abs(p - t)
                dm = jnp.minimum(d, 1.0)
                sq = sq + dm * dm
                mx = mx + jnp.maximum(d, 1.0)
            agz = agz + (0.5 * sq + mx - 2.0) * maskf
            return (abox, alm, agz)

        return lax.fori_loop(0, GRP, group, accs, unroll=2)

    z = jnp.zeros((L,), jnp.float32)
    accs = (z, z, z)
    pending = start(0, 0)
    for c in range(NCHUNK):
        for hd in pending:
            hd.wait()
        if c + 1 < NCHUNK:
            nxt = start(c + 1, (c + 1) % 2)
        else:
            nxt = []
        accs = compute(c % 2, accs)
        pending = nxt

    out_v[0, :] = accs[0]
    out_v[1, :] = accs[1]
    out_v[2, :] = accs[2]
    pltpu.sync_copy(out_v, out_h.at[wid])


_sc_loss = functools.partial(
    pl.kernel,
    out_type=jax.ShapeDtypeStruct((NW, 3, L), jnp.float32),
    mesh=_mesh,
    scratch_types=[
        pltpu.VMEM((4 * CH,), jnp.float32),
        pltpu.VMEM((4 * CH,), jnp.float32),
        pltpu.VMEM((10 * CH,), jnp.float32),
        pltpu.VMEM((10 * CH,), jnp.float32),
        pltpu.VMEM((8, GT * 256), jnp.float32),
        pltpu.VMEM((8, GT * 256), jnp.float32),
        pltpu.VMEM((CH,), jnp.int32),
        pltpu.VMEM((CH,), jnp.int32),
        pltpu.VMEM((CH,), jnp.float32),
        pltpu.VMEM((CH,), jnp.float32),
        pltpu.VMEM((8, 4 * M), jnp.float32),
        pltpu.VMEM((8, 10 * M), jnp.float32),
        pltpu.VMEM((8, 2 * M), jnp.float32),
        pltpu.VMEM((3, L), jnp.float32),
        pltpu.SemaphoreType.DMA,
        pltpu.SemaphoreType.DMA,
    ],
    compiler_params=pltpu.CompilerParams(
        needs_layout_passes=False, use_tc_tiling_on_sc=False),
)(_body)


def _tile_view(x):
    c = x.shape[2]
    return (x.transpose(2, 0, 1)
             .reshape(c, 2, 8, 128, 128)
             .transpose(0, 1, 3, 2, 4)
             .reshape(c, 2, TRW))


def _tile_view2(x):
    return (x.reshape(2, 8, 128, 128)
             .transpose(0, 2, 1, 3)
             .reshape(2, TRW))


def _bce_body(x_ref, lab_ref, out_ref):
    x = x_ref[...]
    lab = lab_ref[...]
    v = jnp.maximum(x, 0.0) - x * lab + jnp.log1p(jnp.exp(-jnp.abs(x)))
    out_ref[...] = jnp.sum(v).reshape(1, 1)


_bce_tc = pl.pallas_call(
    _bce_body,
    out_shape=jax.ShapeDtypeStruct((1, 1), jnp.float32),
)


def kernel(face_preds, landmark_preds, gaze_preds, boxes, landmarks, gaze,
           matches, labels):
    bce = _bce_tc(face_preds[:, :, 4], labels)[0, 0]
    part = _sc_loss(
        _tile_view(face_preds),
        _tile_view(landmark_preds),
        gaze_preds.reshape(B, 128, 128, 2).transpose(0, 1, 3, 2).reshape(B, 32768),
        jnp.transpose(boxes, (0, 2, 1)).reshape(B, 4 * M),
        jnp.transpose(landmarks, (0, 2, 1)).reshape(B, 10 * M),
        jnp.transpose(gaze, (0, 2, 1)).reshape(B, 2 * M),
        _tile_view2(matches.astype(jnp.int32)),
        _tile_view2(labels),
    )
    s = jnp.sum(part, axis=(0, 2))
    face_loss = bce + s[0]
    landmark_loss = s[1]
    gaze_loss = s[2]
    total_loss = face_loss + landmark_loss + gaze_loss
    return (total_loss, face_loss, landmark_loss, gaze_loss)


# speedup vs baseline: 1.2491x; 1.2491x over previous
import functools

import jax
import jax.numpy as jnp
from jax import lax
from jax.experimental import pallas as pl
from jax.experimental.pallas import tpu as pltpu
from jax.experimental.pallas import tpu_sc as plsc

B = 16
N = 16384
M = 64
L = 16
NC = 2
NS = 16
NW = NC * NS
TRW = B * N // 2
SPW = TRW // 16
CH = 2048
NCHUNK = SPW // CH
GRP = CH // L
GT = CH // 1024

_mesh = plsc.VectorSubcoreMesh(core_axis_name="c", subcore_axis_name="s")


def _body(face_h, lmp_h, gzp_h, tbox_h, tlm_h, tgz_h, mat_h, lab_h, out_h,
          face_v0, face_v1, lmp_v0, lmp_v1, gzp_v0, gzp_v1,
          mat_v0, mat_v1, lab_v0, lab_v1,
          tbox_v, tlm_v, tgz_v, out_v, sem0, sem1):
    cid = lax.axis_index("c")
    sid = lax.axis_index("s")
    wid = sid * NC + cid
    tr = wid // 16
    stripe = wid % 16
    w0 = stripe * SPW

    pltpu.sync_copy(tbox_h.at[pl.ds(tr * 8, 8)], tbox_v)
    pltpu.sync_copy(tlm_h.at[pl.ds(tr * 8, 8)], tlm_v)
    pltpu.sync_copy(tgz_h.at[pl.ds(tr * 8, 8)], tgz_v)

    bufs = ((face_v0, lmp_v0, gzp_v0, mat_v0, lab_v0, sem0),
            (face_v1, lmp_v1, gzp_v1, mat_v1, lab_v1, sem1))

    def start(c, slot):
        fv, lv, gv, mv, bv, sem = bufs[slot]
        base = w0 + c * CH
        gz0 = (base // 1024) * 256
        hs = []
        for j in range(4):
            hs.append(pltpu.async_copy(
                face_h.at[j, tr, pl.ds(base, CH)], fv.at[pl.ds(j * CH, CH)], sem))
        for j in range(10):
            hs.append(pltpu.async_copy(
                lmp_h.at[j, tr, pl.ds(base, CH)], lv.at[pl.ds(j * CH, CH)], sem))
        hs.append(pltpu.async_copy(
            gzp_h.at[pl.ds(tr * 8, 8), pl.ds(gz0, GT * 256)], gv, sem))
        hs.append(pltpu.async_copy(mat_h.at[tr, pl.ds(base, CH)], mv, sem))
        hs.append(pltpu.async_copy(lab_h.at[tr, pl.ds(base, CH)], bv, sem))
        return hs

    iota = jnp.arange(L, dtype=jnp.int32)
    izero = iota * 0
    if5 = [iota + j * CH for j in range(4)]
    if10 = [iota + j * CH for j in range(10)]
    it4 = [izero + j * M for j in range(4)]
    it10 = [izero + j * M for j in range(10)]
    it2 = [izero + j * M for j in range(2)]
    ig2 = [iota + j * 128 for j in range(2)]

    def compute(slot, accs):
        fv, lv, gv, mv, bv, _ = bufs[slot]

        def group(g, accs):
            abox, alm, agz = accs
            off = g * L
            row = (g // 8) % 8
            fvo = fv.at[pl.ds(off, L)]
            lvo = lv.at[pl.ds(off, L)]
            m = plsc.load_gather(mv.at[pl.ds(off, L)], [iota])
            lab = plsc.load_gather(bv.at[pl.ds(off, L)], [iota])
            maskf = jnp.where(lab > 0.0, 1.0, 0.0).astype(jnp.float32)

            def sl1(pred_sl, p_bases, tbl_ref, t_bases, ncomp):
                sq = jnp.zeros((L,), jnp.float32)
                mx = jnp.zeros((L,), jnp.float32)
                for j in range(ncomp):
                    p = plsc.load_gather(pred_sl, [p_bases[j]])
                    t = plsc.load_gather(tbl_ref, [m + t_bases[j]])
                    d = jnp.abs(p - t)
                    dm = jnp.minimum(d, 1.0)
                    sq = sq + dm * dm
                    mx = mx + jnp.maximum(d, 1.0)
                return (0.5 * sq + mx - float(ncomp)) * maskf

            abox = abox + sl1(fvo, if5, tbox_v.at[row], it4, 4)
            alm = alm + sl1(lvo, if10, tlm_v.at[row], it10, 10)

            gb = (g // 64) * 256 + (g % 8) * L
            gvo = gv.at[row, pl.ds(gb, L)]
            sq = jnp.zeros((L,), jnp.float32)
            mx = jnp.zeros((L,), jnp.float32)
            for j in range(2):
                p = plsc.load_gather(gvo, [ig2[j]])
                t = plsc.load_gather(tgz_v.at[row], [m + it2[j]])
                d = jnp.abs(p - t)
                dm = jnp.minimum(d, 1.0)
                sq = sq + dm * dm
                mx = mx + jnp.maximum(d, 1.0)
            agz = agz + (0.5 * sq + mx - 2.0) * maskf
            return (abox, alm, agz)

        return lax.fori_loop(0, GRP, group, accs)

    z = jnp.zeros((L,), jnp.float32)
    accs = (z, z, z)
    pending = start(0, 0)
    for c in range(NCHUNK):
        for hd in pending:
            hd.wait()
        if c + 1 < NCHUNK:
            nxt = start(c + 1, (c + 1) % 2)
        else:
            nxt = []
        accs = compute(c % 2, accs)
        pending = nxt

    out_v[0, :] = accs[0]
    out_v[1, :] = accs[1]
    out_v[2, :] = accs[2]
    pltpu.sync_copy(out_v, out_h.at[wid])


_sc_loss = functools.partial(
    pl.kernel,
    out_type=jax.ShapeDtypeStruct((NW, 3, L), jnp.float32),
    mesh=_mesh,
    scratch_types=[
        pltpu.VMEM((4 * CH,), jnp.float32),
        pltpu.VMEM((4 * CH,), jnp.float32),
        pltpu.VMEM((10 * CH,), jnp.float32),
        pltpu.VMEM((10 * CH,), jnp.float32),
        pltpu.VMEM((8, GT * 256), jnp.float32),
        pltpu.VMEM((8, GT * 256), jnp.float32),
        pltpu.VMEM((CH,), jnp.int32),
        pltpu.VMEM((CH,), jnp.int32),
        pltpu.VMEM((CH,), jnp.float32),
        pltpu.VMEM((CH,), jnp.float32),
        pltpu.VMEM((8, 4 * M), jnp.float32),
        pltpu.VMEM((8, 10 * M), jnp.float32),
        pltpu.VMEM((8, 2 * M), jnp.float32),
        pltpu.VMEM((3, L), jnp.float32),
        pltpu.SemaphoreType.DMA,
        pltpu.SemaphoreType.DMA,
    ],
    compiler_params=pltpu.CompilerParams(
        needs_layout_passes=False, use_tc_tiling_on_sc=False),
)(_body)


def _tile_view(x):
    c = x.shape[2]
    return (x.transpose(2, 0, 1)
             .reshape(c, 2, 8, 128, 128)
             .transpose(0, 1, 3, 2, 4)
             .reshape(c, 2, TRW))


def _tile_view2(x):
    return (x.reshape(2, 8, 128, 128)
             .transpose(0, 2, 1, 3)
             .reshape(2, TRW))


def _bce_body(x_ref, lab_ref, out_ref):
    x = x_ref[...]
    lab = lab_ref[...]
    v = jnp.maximum(x, 0.0) - x * lab + jnp.log1p(jnp.exp(-jnp.abs(x)))
    out_ref[...] = jnp.sum(v).reshape(1, 1)


_bce_tc = pl.pallas_call(
    _bce_body,
    out_shape=jax.ShapeDtypeStruct((1, 1), jnp.float32),
)


def kernel(face_preds, landmark_preds, gaze_preds, boxes, landmarks, gaze,
           matches, labels):
    bce = _bce_tc(face_preds[:, :, 4], labels)[0, 0]
    part = _sc_loss(
        _tile_view(face_preds),
        _tile_view(landmark_preds),
        gaze_preds.reshape(B, 128, 128, 2).transpose(0, 1, 3, 2).reshape(B, 32768),
        jnp.transpose(boxes, (0, 2, 1)).reshape(B, 4 * M),
        jnp.transpose(landmarks, (0, 2, 1)).reshape(B, 10 * M),
        jnp.transpose(gaze, (0, 2, 1)).reshape(B, 2 * M),
        _tile_view2(matches.astype(jnp.int32)),
        _tile_view2(labels),
    )
    s = jnp.sum(part, axis=(0, 2))
    face_loss = bce + s[0]
    landmark_loss = s[1]
    gaze_loss = s[2]
    total_loss = face_loss + landmark_loss + gaze_loss
    return (total_loss, face_loss, landmark_loss, gaze_loss)
